# Initial kernel scaffold; baseline (speedup 1.0000x reference)
import functools

import jax
import jax.numpy as jnp
from jax import lax
from jax.experimental import pallas as pl
from jax.experimental.pallas import tpu as pltpu
from jax.experimental.pallas import tpu_sc as plsc

N = 10000
E = 160000
DC = 128
NS = 16
NCORE = 2
CHUNK = 128
KT = 79
EPAD = NS * KT * CHUNK
NACC = 10016
ZROW = NACC // NS
OROW = N // NS


def _spmm_body(nblk, with_deg, *refs):
    if with_deg:
        (x_hbm, src_hbm, dst_hbm, zeros_hbm, out_hbm, deg_hbm,
         src_v, dst_v, xrow_v, rows_v, ones_v, acc, accdeg, sem) = refs
    else:
        (x_hbm, src_hbm, dst_hbm, zeros_hbm, out_hbm,
         src_v, dst_v, xrow_v, rows_v, acc, sem) = refs

    c = lax.axis_index("c")
    s = lax.axis_index("s")

    pltpu.sync_copy(src_hbm.at[pl.ds(s * KT, KT), :], src_v)
    pltpu.sync_copy(dst_hbm.at[pl.ds(s * KT, KT), :], dst_v)

    if with_deg:
        def _fill_ones(i, carry):
            ones_v[i, :] = jnp.ones((16,), jnp.float32)
            return carry
        lax.fori_loop(0, CHUNK, _fill_ones, 0)

    bps = nblk // NCORE
    for bb in range(bps):
        blk = c * bps + bb

        pltpu.sync_copy(zeros_hbm.at[pl.ds(s * ZROW, ZROW), :],
                        acc.at[pl.ds(s * ZROW, ZROW), :])
        if with_deg and bb == 0:
            @pl.when(c == 0)
            def _():
                pltpu.sync_copy(zeros_hbm.at[pl.ds(s * ZROW, ZROW), pl.ds(0, 16)],
                                accdeg.at[pl.ds(s * ZROW, ZROW), :])
        plsc.subcore_barrier()

        def _xrow(j, carry):
            for u in range(CHUNK // 16):
                sl = pl.ds(u * 16, 16)
                xrow_v[j, sl] = src_v[j, sl] * nblk + blk
            return carry
        lax.fori_loop(0, KT, _xrow, 0)

        def _chunk(j, carry):
            pltpu.async_copy(x_hbm.at[xrow_v.at[j]], rows_v, sem).wait()
            pltpu.sync_copy(rows_v, acc.at[dst_v.at[j]], add=True)
            return carry
        lax.fori_loop(0, KT, _chunk, 0)

        if with_deg and bb == 0:
            @pl.when(c == 0)
            def _():
                def _deg(j, carry):
                    pltpu.sync_copy(ones_v, accdeg.at[dst_v.at[j]], add=True)
                    return carry
                lax.fori_loop(0, KT, _deg, 0)

        plsc.subcore_barrier()

        pltpu.sync_copy(acc.at[pl.ds(s * OROW, OROW), :],
                        out_hbm.at[pl.ds(s * OROW, OROW),
                                   pl.ds(blk * DC, DC)])
        if with_deg and bb == 0:
            @pl.when(c == 0)
            def _():
                pltpu.sync_copy(accdeg.at[pl.ds(s * OROW, OROW), :],
                                deg_hbm.at[pl.ds(s * OROW, OROW), :])
        plsc.subcore_barrier()


def _make_spmm(nblk, with_deg):
    d = nblk * DC
    out_type = [jax.ShapeDtypeStruct((N, d), jnp.float32)]
    if with_deg:
        out_type.append(jax.ShapeDtypeStruct((N, 16), jnp.float32))
    scratch = [
        pltpu.VMEM((KT, CHUNK), jnp.int32),
        pltpu.VMEM((KT, CHUNK), jnp.int32),
        pltpu.VMEM((KT, CHUNK), jnp.int32),
        pltpu.VMEM((CHUNK, DC), jnp.float32),
    ]
    if with_deg:
        scratch.append(pltpu.VMEM((CHUNK, 16), jnp.float32))
    scratch.append(pltpu.VMEM_SHARED((NACC, DC), jnp.float32))
    if with_deg:
        scratch.append(pltpu.VMEM_SHARED((NACC, 16), jnp.float32))
    scratch.append(pltpu.SemaphoreType.DMA)
    mesh = plsc.VectorSubcoreMesh(core_axis_name="c", subcore_axis_name="s")
    return pl.kernel(
        functools.partial(_spmm_body, nblk, with_deg),
        out_type=tuple(out_type),
        mesh=mesh,
        scratch_types=scratch,
    )


RT = 400


def _tc0_body(agg_ref, deg_ref, w0_ref, b0_ref, w1_ref, h0_ref, z1_ref):
    d = jnp.maximum(deg_ref[:, 0:1], 1.0)
    x = agg_ref[...] / d
    h0 = jnp.dot(x, w0_ref[...], preferred_element_type=jnp.float32)
    h0 = jnp.maximum(h0 + b0_ref[...], 0.0)
    h0_ref[...] = h0
    z1_ref[...] = jnp.dot(h0, w1_ref[...], preferred_element_type=jnp.float32)


def _tc1_body(agg_ref, deg_ref, b1_ref, h0_ref, wo_ref, z2_ref):
    d = jnp.maximum(deg_ref[:, 0:1], 1.0)
    h = jnp.maximum(agg_ref[...] / d + b1_ref[...], 0.0) + h0_ref[...]
    z2_ref[...] = jnp.dot(h, wo_ref[...], preferred_element_type=jnp.float32)


def _tc2_body(agg_ref, deg_ref, bo_ref, out_ref):
    d = jnp.maximum(deg_ref[:, 0:1], 1.0)
    out_ref[...] = agg_ref[...] / d + bo_ref[...]


def _row_spec(cols):
    return pl.BlockSpec((RT, cols), lambda r: (r, 0))


def _full_spec(rows, cols):
    return pl.BlockSpec((rows, cols), lambda r: (0, 0))


def _tc0(agg0, deg, W0, b0, W1):
    return pl.pallas_call(
        _tc0_body,
        grid=(N // RT,),
        in_specs=[_row_spec(256), _row_spec(16), _full_spec(256, 512),
                  _full_spec(1, 512), _full_spec(512, 512)],
        out_specs=[_row_spec(512), _row_spec(512)],
        out_shape=[jax.ShapeDtypeStruct((N, 512), jnp.float32),
                   jax.ShapeDtypeStruct((N, 512), jnp.float32)],
    )(agg0, deg, W0, b0, W1)


def _tc1(agg1, deg, b1, h0, Wo):
    return pl.pallas_call(
        _tc1_body,
        grid=(N // RT,),
        in_specs=[_row_spec(512), _row_spec(16), _full_spec(1, 512),
                  _row_spec(512), _full_spec(512, 256)],
        out_specs=_row_spec(256),
        out_shape=jax.ShapeDtypeStruct((N, 256), jnp.float32),
    )(agg1, deg, b1, h0, Wo)


def _tc2(agg2, deg, bo):
    return pl.pallas_call(
        _tc2_body,
        grid=(N // RT,),
        in_specs=[_row_spec(256), _row_spec(16), _full_spec(1, 256)],
        out_specs=_row_spec(256),
        out_shape=jax.ShapeDtypeStruct((N, 256), jnp.float32),
    )(agg2, deg, bo)


def kernel(feats, edge_index, W0, b0, W1, b1, Wo, bo):
    src = edge_index[0]
    dst = edge_index[1]
    pad = EPAD - E
    src2 = jnp.concatenate([src, jnp.zeros((pad,), jnp.int32)]).reshape(NS * KT, CHUNK)
    dst2 = jnp.concatenate([dst, jnp.full((pad,), N, jnp.int32)]).reshape(NS * KT, CHUNK)
    zeros = jnp.zeros((NACC, DC), jnp.float32)

    agg0, deg = _make_spmm(2, True)(feats.reshape(N * 2, DC), src2, dst2, zeros)
    h0, z1 = _tc0(agg0, deg, W0, b0.reshape(1, -1), W1)
    (agg1,) = _make_spmm(4, False)(z1.reshape(N * 4, DC), src2, dst2, zeros)
    z2 = _tc1(agg1, deg, b1.reshape(1, -1), h0, Wo)
    (agg2,) = _make_spmm(2, False)(z2.reshape(N * 2, DC), src2, dst2, zeros)
    return _tc2(agg2, deg, bo.reshape(1, -1))


# speedup vs baseline: 4.3214x; 4.3214x over previous
import functools

import jax
import jax.numpy as jnp
from jax import lax
from jax.experimental import pallas as pl
from jax.experimental.pallas import tpu as pltpu
from jax.experimental.pallas import tpu_sc as plsc

N = 10000
E = 160000
DC = 128
NS = 16
NCORE = 2
CHUNK = 128
KT = 79
EPAD = NS * KT * CHUNK
NACC = 10112
ZROW = NACC // NS


def _spmm_body(nblk, with_deg, *refs):
    if with_deg:
        (x_hbm, src_hbm, dst_hbm, zeros_hbm, out_hbm, deg_hbm,
         src_v, dst_v, xrow_v, rows_v, acc, sem) = refs
    else:
        (x_hbm, src_hbm, dst_hbm, zeros_hbm, out_hbm,
         src_v, dst_v, xrow_v, rows_v, acc, sem) = refs

    c = lax.axis_index("c")
    s = lax.axis_index("s")
    r0 = pl.multiple_of(s * ZROW, 8)

    pltpu.sync_copy(src_hbm.at[s], src_v)
    pltpu.sync_copy(dst_hbm.at[s], dst_v)

    bps = nblk // NCORE
    for bb in range(bps):
        blk = c * bps + bb
        c0 = pl.multiple_of(blk * DC, DC)

        pltpu.sync_copy(zeros_hbm.at[pl.ds(r0, ZROW), :],
                        acc.at[pl.ds(r0, ZROW), :])
        plsc.subcore_barrier()

        def _xrow(j, carry):
            for u in range(CHUNK // 16):
                sl = pl.ds(u * 16, 16)
                xrow_v[j, sl] = src_v[j, sl] * nblk + blk
            return carry
        lax.fori_loop(0, KT, _xrow, 0)

        def _chunk(j, carry):
            pltpu.async_copy(x_hbm.at[xrow_v.at[j]], rows_v, sem).wait()
            pltpu.sync_copy(rows_v, acc.at[dst_v.at[j]], add=True)
            return carry
        lax.fori_loop(0, KT, _chunk, 0)
        plsc.subcore_barrier()

        pltpu.sync_copy(acc.at[pl.ds(r0, ZROW), :],
                        out_hbm.at[pl.ds(r0, ZROW), pl.ds(c0, DC)])
        plsc.subcore_barrier()

    if with_deg:
        @pl.when(c == 0)
        def _():
            pltpu.sync_copy(zeros_hbm.at[pl.ds(r0, ZROW), :],
                            acc.at[pl.ds(r0, ZROW), :])

            def _fill_ones(i, carry):
                for u in range(DC // 16):
                    rows_v[i, pl.ds(u * 16, 16)] = jnp.ones((16,), jnp.float32)
                return carry
            lax.fori_loop(0, CHUNK, _fill_ones, 0)
        plsc.subcore_barrier()

        @pl.when(c == 0)
        def _():
            def _deg(j, carry):
                pltpu.sync_copy(rows_v, acc.at[dst_v.at[j]], add=True)
                return carry
            lax.fori_loop(0, KT, _deg, 0)
        plsc.subcore_barrier()

        @pl.when(c == 0)
        def _():
            pltpu.sync_copy(acc.at[pl.ds(r0, ZROW), :],
                            deg_hbm.at[pl.ds(r0, ZROW), :])
        plsc.subcore_barrier()


def _make_spmm(nblk, with_deg):
    d = nblk * DC
    out_type = [jax.ShapeDtypeStruct((NACC, d), jnp.float32)]
    if with_deg:
        out_type.append(jax.ShapeDtypeStruct((NACC, DC), jnp.float32))
    scratch = [
        pltpu.VMEM((KT, CHUNK), jnp.int32),
        pltpu.VMEM((KT, CHUNK), jnp.int32),
        pltpu.VMEM((KT, CHUNK), jnp.int32),
        pltpu.VMEM((CHUNK, DC), jnp.float32),
        pltpu.VMEM_SHARED((NACC, DC), jnp.float32),
        pltpu.SemaphoreType.DMA,
    ]
    mesh = plsc.VectorSubcoreMesh(core_axis_name="c", subcore_axis_name="s")
    return pl.kernel(
        functools.partial(_spmm_body, nblk, with_deg),
        out_type=tuple(out_type),
        mesh=mesh,
        scratch_types=scratch,
    )


RT = 400


def _tc0_body(agg_ref, deg_ref, w0_ref, b0_ref, w1_ref, h0_ref, z1_ref):
    d = jnp.maximum(deg_ref[:, 0:1], 1.0)
    x = agg_ref[...] / d
    h0 = jnp.dot(x, w0_ref[...], preferred_element_type=jnp.float32)
    h0 = jnp.maximum(h0 + b0_ref[...], 0.0)
    h0_ref[...] = h0
    z1_ref[...] = jnp.dot(h0, w1_ref[...], preferred_element_type=jnp.float32)


def _tc1_body(agg_ref, deg_ref, b1_ref, h0_ref, wo_ref, z2_ref):
    d = jnp.maximum(deg_ref[:, 0:1], 1.0)
    h = jnp.maximum(agg_ref[...] / d + b1_ref[...], 0.0) + h0_ref[...]
    z2_ref[...] = jnp.dot(h, wo_ref[...], preferred_element_type=jnp.float32)


def _tc2_body(agg_ref, deg_ref, bo_ref, out_ref):
    d = jnp.maximum(deg_ref[:, 0:1], 1.0)
    out_ref[...] = agg_ref[...] / d + bo_ref[...]


def _row_spec(cols):
    return pl.BlockSpec((RT, cols), lambda r: (r, 0))


def _full_spec(rows, cols):
    return pl.BlockSpec((rows, cols), lambda r: (0, 0))


def _tc0(agg0, deg, W0, b0, W1):
    return pl.pallas_call(
        _tc0_body,
        grid=(N // RT,),
        in_specs=[_row_spec(256), _row_spec(DC), _full_spec(256, 512),
                  _full_spec(1, 512), _full_spec(512, 512)],
        out_specs=[_row_spec(512), _row_spec(512)],
        out_shape=[jax.ShapeDtypeStruct((N, 512), jnp.float32),
                   jax.ShapeDtypeStruct((N, 512), jnp.float32)],
    )(agg0, deg, W0, b0, W1)


def _tc1(agg1, deg, b1, h0, Wo):
    return pl.pallas_call(
        _tc1_body,
        grid=(N // RT,),
        in_specs=[_row_spec(512), _row_spec(DC), _full_spec(1, 512),
                  _row_spec(512), _full_spec(512, 256)],
        out_specs=_row_spec(256),
        out_shape=jax.ShapeDtypeStruct((N, 256), jnp.float32),
    )(agg1, deg, b1, h0, Wo)


def _tc2(agg2, deg, bo):
    return pl.pallas_call(
        _tc2_body,
        grid=(N // RT,),
        in_specs=[_row_spec(256), _row_spec(DC), _full_spec(1, 256)],
        out_specs=_row_spec(256),
        out_shape=jax.ShapeDtypeStruct((N, 256), jnp.float32),
    )(agg2, deg, bo)


def kernel(feats, edge_index, W0, b0, W1, b1, Wo, bo):
    src = edge_index[0]
    dst = edge_index[1]
    pad = EPAD - E
    src2 = jnp.concatenate([src, jnp.zeros((pad,), jnp.int32)]).reshape(NS, KT, CHUNK)
    dst2 = jnp.concatenate([dst, jnp.full((pad,), N, jnp.int32)]).reshape(NS, KT, CHUNK)
    zeros = jnp.zeros((NACC, DC), jnp.float32)

    agg0, deg = _make_spmm(2, True)(
        feats.reshape(N * 2, DC), src2, dst2, zeros)
    h0, z1 = _tc0(agg0, deg, W0, b0.reshape(1, -1), W1)
    (agg1,) = _make_spmm(4, False)(z1.reshape(N * 4, DC), src2, dst2, zeros)
    z2 = _tc1(agg1, deg, b1.reshape(1, -1), h0, Wo)
    (agg2,) = _make_spmm(2, False)(z2.reshape(N * 2, DC), src2, dst2, zeros)
    return _tc2(agg2, deg, bo.reshape(1, -1))
